# baseline (device time: 94533 ns/iter reference)
import jax
import jax.numpy as jnp
from jax import lax
from jax.experimental import pallas as pl
from jax.experimental.pallas import tpu as pltpu

N_DEV = 4
B = 512
H = 512
CHUNK = B // N_DEV
N_LAYERS = 3
HOPS_PER_LAYER = 2 * (N_DEV - 1)
N_HOPS = N_LAYERS * HOPS_PER_LAYER


def kernel(x, Win0, Wout0, Win1, Wout1, Win2, Wout2):
    b, d_shard = x.shape

    def body(x_ref, win0_ref, wout0_ref, win1_ref, wout1_ref,
             win2_ref, wout2_ref, out_ref,
             h_ref, hr_ref, rs_buf, send_sems, recv_sems):
        my_pos = lax.axis_index("i")
        left = (my_pos - 1) % N_DEV
        right = (my_pos + 1) % N_DEV

        barrier_sem = pltpu.get_barrier_semaphore()
        for nbr in (left, right):
            pl.semaphore_signal(
                barrier_sem, inc=1,
                device_id=(nbr,), device_id_type=pl.DeviceIdType.MESH,
            )
        pl.semaphore_wait(barrier_sem, 2)

        def all_reduce_relu(layer):
            for s in range(N_DEV - 1):
                hop = layer * HOPS_PER_LAYER + s
                send_chunk = (my_pos - s) % N_DEV
                rdma = pltpu.make_async_remote_copy(
                    src_ref=h_ref.at[pl.ds(send_chunk * CHUNK, CHUNK), :],
                    dst_ref=rs_buf.at[s],
                    send_sem=send_sems.at[hop],
                    recv_sem=recv_sems.at[hop],
                    device_id=(right,),
                    device_id_type=pl.DeviceIdType.MESH,
                )
                rdma.start()
                rdma.wait()
                add_chunk = (my_pos - s - 1) % N_DEV
                h_ref[pl.ds(add_chunk * CHUNK, CHUNK), :] = (
                    h_ref[pl.ds(add_chunk * CHUNK, CHUNK), :] + rs_buf[s]
                )

            mine = (my_pos + 1) % N_DEV
            hr_ref[pl.ds(mine * CHUNK, CHUNK), :] = jnp.maximum(
                h_ref[pl.ds(mine * CHUNK, CHUNK), :], 0.0
            )

            for s in range(N_DEV - 1):
                hop = layer * HOPS_PER_LAYER + (N_DEV - 1) + s
                send_chunk = (my_pos + 1 - s) % N_DEV
                rdma = pltpu.make_async_remote_copy(
                    src_ref=hr_ref.at[pl.ds(send_chunk * CHUNK, CHUNK), :],
                    dst_ref=hr_ref.at[pl.ds(send_chunk * CHUNK, CHUNK), :],
                    send_sem=send_sems.at[hop],
                    recv_sem=recv_sems.at[hop],
                    device_id=(right,),
                    device_id_type=pl.DeviceIdType.MESH,
                )
                rdma.start()
                rdma.wait()

        x_val = x_ref[:, :]
        for layer, (win_ref, wout_ref) in enumerate(
            [(win0_ref, wout0_ref), (win1_ref, wout1_ref), (win2_ref, wout2_ref)]
        ):
            h_ref[:, :] = jnp.dot(
                x_val, win_ref[:, :], preferred_element_type=jnp.float32
            )
            all_reduce_relu(layer)
            x_val = jnp.dot(
                hr_ref[:, :], wout_ref[:, :], preferred_element_type=jnp.float32
            )

        out_ref[:, :] = x_val

    return pl.pallas_call(
        body,
        out_shape=jax.ShapeDtypeStruct((b, d_shard), jnp.float32),
        in_specs=[pl.BlockSpec(memory_space=pltpu.VMEM)] * 7,
        out_specs=pl.BlockSpec(memory_space=pltpu.VMEM),
        scratch_shapes=[
            pltpu.VMEM((B, H), jnp.float32),
            pltpu.VMEM((B, H), jnp.float32),
            pltpu.VMEM((N_DEV - 1, CHUNK, H), jnp.float32),
            pltpu.SemaphoreType.DMA((N_HOPS,)),
            pltpu.SemaphoreType.DMA((N_HOPS,)),
        ],
        compiler_params=pltpu.CompilerParams(collective_id=0),
    )(x, Win0, Wout0, Win1, Wout1, Win2, Wout2)


# device time: 54853 ns/iter; 1.7234x vs baseline; 1.7234x over previous
import jax
import jax.numpy as jnp
from jax import lax
from jax.experimental import pallas as pl
from jax.experimental.pallas import tpu as pltpu

N_DEV = 4
B = 512
H = 512
HALF = 128
N_LAYERS = 3
N_SEMS = N_LAYERS * 6


def kernel(x, Win0, Wout0, Win1, Wout1, Win2, Wout2):
    b, d_shard = x.shape

    def body(x_ref, win0_ref, wout0_ref, win1_ref, wout1_ref,
             win2_ref, wout2_ref, out_ref,
             h_ref, hr_ref, st1_buf, st2_buf, send_sems, recv_sems):
        my_pos = lax.axis_index("i")
        pA = my_pos ^ 1
        pB = 3 - my_pos
        k1T = (my_pos ^ (my_pos >> 1)) & 1
        k1U = my_pos >> 1

        tT_keep = k1T * HALF
        tT_send = (1 - k1T) * HALF
        uU_keep = 2 * HALF + k1U * HALF
        uU_send = 2 * HALF + (1 - k1U) * HALF

        barrier_sem = pltpu.get_barrier_semaphore()
        for nbr in (pA, pB):
            pl.semaphore_signal(
                barrier_sem, inc=1,
                device_id=(nbr,), device_id_type=pl.DeviceIdType.MESH,
            )
        pl.semaphore_wait(barrier_sem, 2)

        def exchange(src_ref, dst_ref, sem_idx, partner):
            return pltpu.make_async_remote_copy(
                src_ref=src_ref,
                dst_ref=dst_ref,
                send_sem=send_sems.at[sem_idx],
                recv_sem=recv_sems.at[sem_idx],
                device_id=(partner,),
                device_id_type=pl.DeviceIdType.MESH,
            )

        def all_reduce_relu(layer):
            base = layer * 6

            rT = exchange(h_ref.at[pl.ds(tT_send, HALF), :], st1_buf.at[0],
                          base + 0, pA)
            rU = exchange(h_ref.at[pl.ds(uU_send, HALF), :], st1_buf.at[1],
                          base + 1, pB)
            rT.start()
            rU.start()
            rT.wait()
            rU.wait()
            h_ref[pl.ds(tT_keep, HALF), :] = (
                h_ref[pl.ds(tT_keep, HALF), :] + st1_buf[0]
            )
            h_ref[pl.ds(uU_keep, HALF), :] = (
                h_ref[pl.ds(uU_keep, HALF), :] + st1_buf[1]
            )

            rT = exchange(h_ref.at[pl.ds(tT_keep, HALF), :], st2_buf.at[0],
                          base + 2, pB)
            rU = exchange(h_ref.at[pl.ds(uU_keep, HALF), :], st2_buf.at[1],
                          base + 3, pA)
            rT.start()
            rU.start()
            rT.wait()
            rU.wait()
            hr_ref[pl.ds(tT_keep, HALF), :] = jnp.maximum(
                h_ref[pl.ds(tT_keep, HALF), :] + st2_buf[0], 0.0
            )
            hr_ref[pl.ds(uU_keep, HALF), :] = jnp.maximum(
                h_ref[pl.ds(uU_keep, HALF), :] + st2_buf[1], 0.0
            )

            rT = exchange(hr_ref.at[pl.ds(tT_keep, HALF), :],
                          hr_ref.at[pl.ds(tT_keep, HALF), :], base + 4, pA)
            rU = exchange(hr_ref.at[pl.ds(uU_keep, HALF), :],
                          hr_ref.at[pl.ds(uU_keep, HALF), :], base + 5, pB)
            rT.start()
            rU.start()
            rT.wait()
            rU.wait()

        x_val = x_ref[:, :]
        for layer, (win_ref, wout_ref) in enumerate(
            [(win0_ref, wout0_ref), (win1_ref, wout1_ref), (win2_ref, wout2_ref)]
        ):
            h_ref[:, :] = jnp.dot(
                x_val, win_ref[:, :], preferred_element_type=jnp.float32
            )
            all_reduce_relu(layer)
            x_val = jnp.dot(
                hr_ref[:, :], wout_ref[:, :], preferred_element_type=jnp.float32
            )

        out_ref[:, :] = x_val

    return pl.pallas_call(
        body,
        out_shape=jax.ShapeDtypeStruct((b, d_shard), jnp.float32),
        in_specs=[pl.BlockSpec(memory_space=pltpu.VMEM)] * 7,
        out_specs=pl.BlockSpec(memory_space=pltpu.VMEM),
        scratch_shapes=[
            pltpu.VMEM((B, H), jnp.float32),
            pltpu.VMEM((B, H), jnp.float32),
            pltpu.VMEM((2, HALF, H), jnp.float32),
            pltpu.VMEM((2, HALF, H), jnp.float32),
            pltpu.SemaphoreType.DMA((N_SEMS,)),
            pltpu.SemaphoreType.DMA((N_SEMS,)),
        ],
        compiler_params=pltpu.CompilerParams(collective_id=0),
    )(x, Win0, Wout0, Win1, Wout1, Win2, Wout2)


# device time: 41400 ns/iter; 2.2834x vs baseline; 1.3250x over previous
import jax
import jax.numpy as jnp
from jax import lax
from jax.experimental import pallas as pl
from jax.experimental.pallas import tpu as pltpu

N_DEV = 4
B = 512
H = 512
HALF = 128
N_LAYERS = 3
N_SEMS = N_LAYERS * 6

F32 = jnp.float32
BF16 = jnp.bfloat16


def kernel(x, Win0, Wout0, Win1, Wout1, Win2, Wout2):
    b, d_shard = x.shape

    def body(x_ref, win0_ref, wout0_ref, win1_ref, wout1_ref,
             win2_ref, wout2_ref, out_ref,
             h_ref, xn_ref, s1s, s1r, s2s, s2r, s3s, s3r,
             send_sems, recv_sems):
        my_pos = lax.axis_index("i")
        pA = my_pos ^ 1
        pB = 3 - my_pos
        k1T = (my_pos ^ (my_pos >> 1)) & 1
        k1U = my_pos >> 1

        tK = k1T * HALF
        tS = (1 - k1T) * HALF
        uK = 2 * HALF + k1U * HALF
        uS = 2 * HALF + (1 - k1U) * HALF

        barrier_sem = pltpu.get_barrier_semaphore()
        for nbr in (pA, pB):
            pl.semaphore_signal(
                barrier_sem, inc=1,
                device_id=(nbr,), device_id_type=pl.DeviceIdType.MESH,
            )
        pl.semaphore_wait(barrier_sem, 2)

        def exch(src_ref, dst_ref, sem_idx, partner):
            return pltpu.make_async_remote_copy(
                src_ref=src_ref,
                dst_ref=dst_ref,
                send_sem=send_sems.at[sem_idx],
                recv_sem=recv_sems.at[sem_idx],
                device_id=(partner,),
                device_id_type=pl.DeviceIdType.MESH,
            )

        win_refs = [win0_ref, win1_ref, win2_ref]
        wout_refs = [wout0_ref, wout1_ref, wout2_ref]

        def x_blk(layer, start):
            src = x_ref if layer == 0 else xn_ref
            return src[pl.ds(start, HALF), :]

        for layer in range(N_LAYERS):
            win = win_refs[layer]
            wout = wout_refs[layer]
            base = layer * 6

            h_ref[pl.ds(tS, HALF), :] = jnp.dot(
                x_blk(layer, tS), win[:, :], preferred_element_type=F32)
            h_ref[pl.ds(uS, HALF), :] = jnp.dot(
                x_blk(layer, uS), win[:, :], preferred_element_type=F32)
            s1s[0] = h_ref[pl.ds(tS, HALF), :].astype(BF16)
            s1s[1] = h_ref[pl.ds(uS, HALF), :].astype(BF16)
            rT = exch(s1s.at[0], s1r.at[0], base + 0, pA)
            rU = exch(s1s.at[1], s1r.at[1], base + 1, pB)
            rT.start()
            rU.start()
            h_ref[pl.ds(tK, HALF), :] = jnp.dot(
                x_blk(layer, tK), win[:, :], preferred_element_type=F32)
            h_ref[pl.ds(uK, HALF), :] = jnp.dot(
                x_blk(layer, uK), win[:, :], preferred_element_type=F32)
            rT.wait()
            rU.wait()
            h_ref[pl.ds(tK, HALF), :] = (
                h_ref[pl.ds(tK, HALF), :] + s1r[0].astype(F32))
            h_ref[pl.ds(uK, HALF), :] = (
                h_ref[pl.ds(uK, HALF), :] + s1r[1].astype(F32))

            s2s[0] = h_ref[pl.ds(tK, HALF), :].astype(BF16)
            s2s[1] = h_ref[pl.ds(uK, HALF), :].astype(BF16)
            rT = exch(s2s.at[0], s2r.at[0], base + 2, pB)
            rU = exch(s2s.at[1], s2r.at[1], base + 3, pA)
            rT.start()
            rU.start()
            rT.wait()
            rU.wait()
            relu_t = jnp.maximum(
                h_ref[pl.ds(tK, HALF), :] + s2r[0].astype(F32), 0.0)
            relu_u = jnp.maximum(
                h_ref[pl.ds(uK, HALF), :] + s2r[1].astype(F32), 0.0)

            s3s[0] = relu_t.astype(BF16)
            s3s[1] = relu_u.astype(BF16)
            rT = exch(s3s.at[0], s3r.at[0], base + 4, pA)
            rU = exch(s3s.at[1], s3r.at[1], base + 5, pB)
            rT.start()
            rU.start()
            xk_t = jnp.dot(relu_t, wout[:, :], preferred_element_type=F32)
            xk_u = jnp.dot(relu_u, wout[:, :], preferred_element_type=F32)
            rT.wait()
            rU.wait()
            xs_t = jnp.dot(s3r[0].astype(F32), wout[:, :],
                           preferred_element_type=F32)
            xs_u = jnp.dot(s3r[1].astype(F32), wout[:, :],
                           preferred_element_type=F32)

            dst = out_ref if layer == N_LAYERS - 1 else xn_ref
            dst[pl.ds(tK, HALF), :] = xk_t
            dst[pl.ds(uK, HALF), :] = xk_u
            dst[pl.ds(tS, HALF), :] = xs_t
            dst[pl.ds(uS, HALF), :] = xs_u

    return pl.pallas_call(
        body,
        out_shape=jax.ShapeDtypeStruct((b, d_shard), F32),
        in_specs=[pl.BlockSpec(memory_space=pltpu.VMEM)] * 7,
        out_specs=pl.BlockSpec(memory_space=pltpu.VMEM),
        scratch_shapes=[
            pltpu.VMEM((B, H), F32),
            pltpu.VMEM((B, d_shard), F32),
            pltpu.VMEM((2, HALF, H), BF16),
            pltpu.VMEM((2, HALF, H), BF16),
            pltpu.VMEM((2, HALF, H), BF16),
            pltpu.VMEM((2, HALF, H), BF16),
            pltpu.VMEM((2, HALF, H), BF16),
            pltpu.VMEM((2, HALF, H), BF16),
            pltpu.SemaphoreType.DMA((N_SEMS,)),
            pltpu.SemaphoreType.DMA((N_SEMS,)),
        ],
        compiler_params=pltpu.CompilerParams(collective_id=0),
    )(x, Win0, Wout0, Win1, Wout1, Win2, Wout2)


# device time: 37005 ns/iter; 2.5546x vs baseline; 1.1188x over previous
import jax
import jax.numpy as jnp
from jax import lax
from jax.experimental import pallas as pl
from jax.experimental.pallas import tpu as pltpu

N_DEV = 4
B = 512
H = 512
HALF = 128
N_LAYERS = 3
N_SEMS = N_LAYERS * 6

F32 = jnp.float32
BF16 = jnp.bfloat16


def kernel(x, Win0, Wout0, Win1, Wout1, Win2, Wout2):
    b, d_shard = x.shape

    def body(x_ref, win0_ref, wout0_ref, win1_ref, wout1_ref,
             win2_ref, wout2_ref, out_ref,
             h_ref, s1s, s1r, s2s, s2r, s3s, s3r,
             send_sems, recv_sems):
        my_pos = lax.axis_index("i")
        pA = my_pos ^ 1
        pB = 3 - my_pos
        k1T = (my_pos ^ (my_pos >> 1)) & 1
        k1U = my_pos >> 1

        tK_e = k1T * HALF
        tS_e = (1 - k1T) * HALF
        uK_e = 2 * HALF + k1U * HALF
        uS_e = 2 * HALF + (1 - k1U) * HALF

        def rows(layer):
            if layer % 2 == 0:
                return tK_e, tS_e, uK_e, uS_e
            return tS_e, tK_e, uS_e, uK_e

        barrier_sem = pltpu.get_barrier_semaphore()
        for nbr in (pA, pB):
            pl.semaphore_signal(
                barrier_sem, inc=1,
                device_id=(nbr,), device_id_type=pl.DeviceIdType.MESH,
            )
        pl.semaphore_wait(barrier_sem, 2)

        def exch(src_ref, dst_ref, sem_idx, partner):
            return pltpu.make_async_remote_copy(
                src_ref=src_ref,
                dst_ref=dst_ref,
                send_sem=send_sems.at[sem_idx],
                recv_sem=recv_sems.at[sem_idx],
                device_id=(partner,),
                device_id_type=pl.DeviceIdType.MESH,
            )

        def start_s1(layer, ht_bf16, hu_bf16):
            s1s[0] = ht_bf16
            s1s[1] = hu_bf16
            rT = exch(s1s.at[0], s1r.at[0], layer * 6 + 0, pA)
            rU = exch(s1s.at[1], s1r.at[1], layer * 6 + 1, pB)
            rT.start()
            rU.start()
            return rT, rU

        win_refs = [win0_ref, win1_ref, win2_ref]
        wout_refs = [wout0_ref, wout1_ref, wout2_ref]

        def dot(a, w_ref):
            return jnp.dot(a, w_ref[:, :], preferred_element_type=F32)

        tK, tS, uK, uS = rows(0)
        s1T, s1U = start_s1(
            0,
            dot(x_ref[pl.ds(tS, HALF), :], win0_ref).astype(BF16),
            dot(x_ref[pl.ds(uS, HALF), :], win0_ref).astype(BF16),
        )
        h_ref[pl.ds(tK, HALF), :] = dot(x_ref[pl.ds(tK, HALF), :], win0_ref)
        h_ref[pl.ds(uK, HALF), :] = dot(x_ref[pl.ds(uK, HALF), :], win0_ref)

        for layer in range(N_LAYERS):
            tK, tS, uK, uS = rows(layer)
            wout = wout_refs[layer]
            base = layer * 6

            s1T.wait()
            s1U.wait()
            h2_t = h_ref[pl.ds(tK, HALF), :] + s1r[0].astype(F32)
            h2_u = h_ref[pl.ds(uK, HALF), :] + s1r[1].astype(F32)

            s2s[0] = h2_t.astype(BF16)
            s2s[1] = h2_u.astype(BF16)
            rT = exch(s2s.at[0], s2r.at[0], base + 2, pB)
            rU = exch(s2s.at[1], s2r.at[1], base + 3, pA)
            rT.start()
            rU.start()
            rT.wait()
            rU.wait()
            relu_t = jnp.maximum(h2_t + s2r[0].astype(F32), 0.0)
            relu_u = jnp.maximum(h2_u + s2r[1].astype(F32), 0.0)

            s3s[0] = relu_t.astype(BF16)
            s3s[1] = relu_u.astype(BF16)
            rT = exch(s3s.at[0], s3r.at[0], base + 4, pA)
            rU = exch(s3s.at[1], s3r.at[1], base + 5, pB)
            rT.start()
            rU.start()

            xk_t = dot(relu_t, wout)
            xk_u = dot(relu_u, wout)
            if layer < N_LAYERS - 1:
                win_n = win_refs[layer + 1]
                s1T, s1U = start_s1(
                    layer + 1,
                    dot(xk_t, win_n).astype(BF16),
                    dot(xk_u, win_n).astype(BF16),
                )

            rT.wait()
            rU.wait()
            xs_t = dot(s3r[0].astype(F32), wout)
            xs_u = dot(s3r[1].astype(F32), wout)

            if layer < N_LAYERS - 1:
                win_n = win_refs[layer + 1]
                h_ref[pl.ds(tS, HALF), :] = dot(xs_t, win_n)
                h_ref[pl.ds(uS, HALF), :] = dot(xs_u, win_n)
            else:
                out_ref[pl.ds(tK, HALF), :] = xk_t
                out_ref[pl.ds(uK, HALF), :] = xk_u
                out_ref[pl.ds(tS, HALF), :] = xs_t
                out_ref[pl.ds(uS, HALF), :] = xs_u

    return pl.pallas_call(
        body,
        out_shape=jax.ShapeDtypeStruct((b, d_shard), F32),
        in_specs=[pl.BlockSpec(memory_space=pltpu.VMEM)] * 7,
        out_specs=pl.BlockSpec(memory_space=pltpu.VMEM),
        scratch_shapes=[
            pltpu.VMEM((B, H), F32),
            pltpu.VMEM((2, HALF, H), BF16),
            pltpu.VMEM((2, HALF, H), BF16),
            pltpu.VMEM((2, HALF, H), BF16),
            pltpu.VMEM((2, HALF, H), BF16),
            pltpu.VMEM((2, HALF, H), BF16),
            pltpu.VMEM((2, HALF, H), BF16),
            pltpu.SemaphoreType.DMA((N_SEMS,)),
            pltpu.SemaphoreType.DMA((N_SEMS,)),
        ],
        compiler_params=pltpu.CompilerParams(collective_id=0),
    )(x, Win0, Wout0, Win1, Wout1, Win2, Wout2)
